# baseline (speedup 1.0000x reference)
import functools

import jax
import jax.numpy as jnp
from jax import lax
from jax.experimental import pallas as pl
from jax.experimental.pallas import tpu as pltpu, tpu_sc as plsc

V = 1000
H = 32
DP = 128

_BT = 2048


@functools.cache
def _make_sc_gather(n_tok: int):
    info = plsc.get_sparse_core_info()
    nc, ns = info.num_cores, info.num_subcores
    nw = nc * ns
    assert n_tok % nw == 0
    b_per_w = n_tok // nw
    mesh = plsc.VectorSubcoreMesh(core_axis_name="c", subcore_axis_name="s")

    @functools.partial(
        pl.kernel, mesh=mesh,
        compiler_params=pltpu.CompilerParams(use_tc_tiling_on_sc=False),
        out_type=jax.ShapeDtypeStruct((n_tok, DP), jnp.float32),
        scratch_types=[
            pltpu.VMEM((b_per_w,), jnp.int32),
            pltpu.VMEM((b_per_w, DP), jnp.float32),
            pltpu.SemaphoreType.DMA,
        ],
    )
    def gather_k(idx_hbm, table_hbm, out_hbm, idx_v, rows_v, sem):
        wid = lax.axis_index("s") * nc + lax.axis_index("c")
        base = wid * b_per_w
        pltpu.sync_copy(idx_hbm.at[pl.ds(base, b_per_w)], idx_v)
        pltpu.async_copy(table_hbm.at[idx_v], rows_v, sem).wait()
        pltpu.sync_copy(rows_v, out_hbm.at[pl.ds(base, b_per_w)])

    return gather_k


def _mlp_kernel(x_ref, w1_ref, b1_ref, w2_ref, b2_ref, out_ref):
    h = jnp.maximum(
        jnp.dot(x_ref[...], w1_ref[...], preferred_element_type=jnp.float32,
                precision=lax.Precision.HIGHEST) + b1_ref[...],
        0.0)
    out_ref[...] = jnp.dot(h, w2_ref[...], preferred_element_type=jnp.float32,
                           precision=lax.Precision.HIGHEST) + b2_ref[...]


def kernel(inputs, emb, W1, b1, W2, b2):
    B, L = inputs.shape
    n_tok = B * L
    idx = inputs.reshape(n_tok).astype(jnp.int32)

    emb_p = jnp.pad(emb, ((0, 0), (0, DP - H)))
    w1_p = jnp.pad(W1, ((0, DP - H), (0, 0)))

    x = _make_sc_gather(n_tok)(idx, emb_p)

    n_blocks = n_tok // _BT
    out = pl.pallas_call(
        _mlp_kernel,
        grid=(n_blocks,),
        in_specs=[
            pl.BlockSpec((_BT, DP), lambda g: (g, 0)),
            pl.BlockSpec((DP, H), lambda g: (0, 0)),
            pl.BlockSpec((1, H), lambda g: (0, 0)),
            pl.BlockSpec((H, V), lambda g: (0, 0)),
            pl.BlockSpec((1, V), lambda g: (0, 0)),
        ],
        out_specs=pl.BlockSpec((_BT, V), lambda g: (g, 0)),
        out_shape=jax.ShapeDtypeStruct((n_tok, V), jnp.float32),
    )(x, w1_p, b1.reshape(1, H), W2, b2.reshape(1, V))
    return out.reshape(B, L, V)


# speedup vs baseline: 1.0087x; 1.0087x over previous
import functools

import jax
import jax.numpy as jnp
from jax import lax
from jax.experimental import pallas as pl
from jax.experimental.pallas import tpu as pltpu, tpu_sc as plsc

V = 1000
H = 32
DP = 128

_BT = 2048


@functools.cache
def _make_sc_gather(n_tok: int):
    info = plsc.get_sparse_core_info()
    nc, ns = info.num_cores, info.num_subcores
    nw = nc * ns
    assert n_tok % nw == 0
    b_per_w = n_tok // nw
    mesh = plsc.VectorSubcoreMesh(core_axis_name="c", subcore_axis_name="s")

    @functools.partial(
        pl.kernel, mesh=mesh,
        compiler_params=pltpu.CompilerParams(use_tc_tiling_on_sc=True),
        out_type=jax.ShapeDtypeStruct((n_tok, DP), jnp.float32),
        scratch_types=[
            pltpu.VMEM((b_per_w,), jnp.int32),
            pltpu.VMEM((b_per_w, DP), jnp.float32),
            pltpu.SemaphoreType.DMA,
        ],
    )
    def gather_k(idx_hbm, table_hbm, out_hbm, idx_v, rows_v, sem):
        wid = lax.axis_index("s") * nc + lax.axis_index("c")
        base = wid * b_per_w
        pltpu.sync_copy(idx_hbm.at[pl.ds(base, b_per_w)], idx_v)
        pltpu.async_copy(table_hbm.at[idx_v], rows_v, sem).wait()
        pltpu.sync_copy(rows_v, out_hbm.at[pl.ds(base, b_per_w)])

    return gather_k


def _mlp_kernel(x_ref, w1_ref, b1_ref, w2_ref, b2_ref, out_ref):
    h = jnp.maximum(
        jnp.dot(x_ref[...], w1_ref[...], preferred_element_type=jnp.float32,
                precision=lax.Precision.HIGHEST) + b1_ref[...],
        0.0)
    out_ref[...] = jnp.dot(h, w2_ref[...], preferred_element_type=jnp.float32,
                           precision=lax.Precision.HIGHEST) + b2_ref[...]


def kernel(inputs, emb, W1, b1, W2, b2):
    B, L = inputs.shape
    n_tok = B * L
    idx = inputs.reshape(n_tok).astype(jnp.int32)

    emb_p = jnp.pad(emb, ((0, 0), (0, DP - H)))
    w1_p = jnp.pad(W1, ((0, DP - H), (0, 0)))

    x = _make_sc_gather(n_tok)(idx, emb_p)

    n_blocks = n_tok // _BT
    out = pl.pallas_call(
        _mlp_kernel,
        grid=(n_blocks,),
        in_specs=[
            pl.BlockSpec((_BT, DP), lambda g: (g, 0)),
            pl.BlockSpec((DP, H), lambda g: (0, 0)),
            pl.BlockSpec((1, H), lambda g: (0, 0)),
            pl.BlockSpec((H, V), lambda g: (0, 0)),
            pl.BlockSpec((1, V), lambda g: (0, 0)),
        ],
        out_specs=pl.BlockSpec((_BT, V), lambda g: (g, 0)),
        out_shape=jax.ShapeDtypeStruct((n_tok, V), jnp.float32),
    )(x, w1_p, b1.reshape(1, H), W2, b2.reshape(1, V))
    return out.reshape(B, L, V)


# speedup vs baseline: 1.3076x; 1.2964x over previous
import functools

import jax
import jax.numpy as jnp
from jax import lax
from jax.experimental import pallas as pl
from jax.experimental.pallas import tpu as pltpu, tpu_sc as plsc

V = 1000
H = 32
DP = 128

_BT = 2048


@functools.cache
def _make_sc_gather(n_tok: int):
    info = plsc.get_sparse_core_info()
    nc, ns = info.num_cores, info.num_subcores
    nw = nc * ns
    assert n_tok % nw == 0
    b_per_w = n_tok // nw
    mesh = plsc.VectorSubcoreMesh(core_axis_name="c", subcore_axis_name="s")

    @functools.partial(
        pl.kernel, mesh=mesh,
        compiler_params=pltpu.CompilerParams(use_tc_tiling_on_sc=True),
        out_type=jax.ShapeDtypeStruct((n_tok, DP), jnp.float32),
        scratch_types=[
            pltpu.VMEM((b_per_w,), jnp.int32),
            pltpu.VMEM((b_per_w, DP), jnp.float32),
            pltpu.SemaphoreType.DMA,
        ],
    )
    def gather_k(idx_hbm, table_hbm, out_hbm, idx_v, rows_v, sem):
        wid = lax.axis_index("s") * nc + lax.axis_index("c")
        base = wid * b_per_w
        pltpu.sync_copy(idx_hbm.at[pl.ds(base, b_per_w)], idx_v)
        pltpu.async_copy(table_hbm.at[idx_v], rows_v, sem).wait()
        pltpu.sync_copy(rows_v, out_hbm.at[pl.ds(base, b_per_w)])

    return gather_k


def _pad_kernel(emb_ref, w1_ref, emb_p_ref, w1_p_ref):
    emb_p_ref[...] = jnp.zeros_like(emb_p_ref)
    emb_p_ref[:, :H] = emb_ref[...]
    w1_p_ref[...] = jnp.zeros_like(w1_p_ref)
    w1_p_ref[:H, :] = w1_ref[...]


def _pad_tables(emb, W1):
    return pl.pallas_call(
        _pad_kernel,
        out_shape=(
            jax.ShapeDtypeStruct((V, DP), jnp.float32),
            jax.ShapeDtypeStruct((DP, H), jnp.float32),
        ),
    )(emb, W1)


def _mlp_kernel(x_ref, w1_ref, b1_ref, w2_ref, b2_ref, out_ref):
    h = jnp.maximum(
        jnp.dot(x_ref[...], w1_ref[...], preferred_element_type=jnp.float32)
        + b1_ref[...],
        0.0)
    out_ref[...] = jnp.dot(h, w2_ref[...],
                           preferred_element_type=jnp.float32) + b2_ref[...]


def kernel(inputs, emb, W1, b1, W2, b2):
    B, L = inputs.shape
    n_tok = B * L
    idx = inputs.reshape(n_tok).astype(jnp.int32)

    emb_p, w1_p = _pad_tables(emb, W1)

    x = _make_sc_gather(n_tok)(idx, emb_p)

    n_blocks = n_tok // _BT
    out = pl.pallas_call(
        _mlp_kernel,
        grid=(n_blocks,),
        in_specs=[
            pl.BlockSpec((_BT, DP), lambda g: (g, 0)),
            pl.BlockSpec((DP, H), lambda g: (0, 0)),
            pl.BlockSpec((1, H), lambda g: (0, 0)),
            pl.BlockSpec((H, V), lambda g: (0, 0)),
            pl.BlockSpec((1, V), lambda g: (0, 0)),
        ],
        out_specs=pl.BlockSpec((_BT, V), lambda g: (g, 0)),
        out_shape=jax.ShapeDtypeStruct((n_tok, V), jnp.float32),
    )(x, w1_p, b1.reshape(1, H), W2, b2.reshape(1, V))
    return out.reshape(B, L, V)


# speedup vs baseline: 1.7771x; 1.3590x over previous
import functools

import jax
import jax.numpy as jnp
from jax import lax
from jax.experimental import pallas as pl
from jax.experimental.pallas import tpu as pltpu, tpu_sc as plsc

V = 1000
H = 32
DP = 128
LP = 24

_BB = 128


@functools.cache
def _make_sc_gather(n_rows: int):
    info = plsc.get_sparse_core_info()
    nc, ns = info.num_cores, info.num_subcores
    nw = nc * ns
    assert n_rows % nw == 0
    b_per_w = n_rows // nw
    mesh = plsc.VectorSubcoreMesh(core_axis_name="c", subcore_axis_name="s")

    @functools.partial(
        pl.kernel, mesh=mesh,
        compiler_params=pltpu.CompilerParams(use_tc_tiling_on_sc=True),
        out_type=jax.ShapeDtypeStruct((n_rows, DP), jnp.float32),
        scratch_types=[
            pltpu.VMEM((b_per_w,), jnp.int32),
            pltpu.VMEM((b_per_w, DP), jnp.float32),
            pltpu.SemaphoreType.DMA,
        ],
    )
    def gather_k(idx_hbm, table_hbm, out_hbm, idx_v, rows_v, sem):
        wid = lax.axis_index("s") * nc + lax.axis_index("c")
        base = wid * b_per_w
        pltpu.sync_copy(idx_hbm.at[pl.ds(base, b_per_w)], idx_v)
        pltpu.async_copy(table_hbm.at[idx_v], rows_v, sem).wait()
        pltpu.sync_copy(rows_v, out_hbm.at[pl.ds(base, b_per_w)])

    return gather_k


def _prep_kernel(inp_ref, emb_ref, w1_ref, ipad_ref, emb_p_ref, w1_p_ref):
    b = inp_ref.shape[0]
    filler = jax.lax.broadcasted_iota(jnp.int32, (b, LP), 0) % V
    ipad_ref[...] = jnp.concatenate(
        [inp_ref[...], filler[:, : LP - inp_ref.shape[1]]], axis=1)
    emb_p_ref[...] = jnp.zeros_like(emb_p_ref)
    emb_p_ref[:, :H] = emb_ref[...]
    w1_p_ref[...] = jnp.zeros_like(w1_p_ref)
    w1_p_ref[:H, :] = w1_ref[...]


def _prep(inputs_i32, emb, W1):
    b = inputs_i32.shape[0]
    return pl.pallas_call(
        _prep_kernel,
        out_shape=(
            jax.ShapeDtypeStruct((b, LP), jnp.int32),
            jax.ShapeDtypeStruct((V, DP), jnp.float32),
            jax.ShapeDtypeStruct((DP, H), jnp.float32),
        ),
    )(inputs_i32, emb, W1)


def _mlp_kernel(x_ref, w1_ref, b1_ref, w2_ref, b2_ref, out_ref):
    h = jnp.maximum(
        jnp.dot(x_ref[...], w1_ref[...], preferred_element_type=jnp.float32)
        + b1_ref[...],
        0.0)
    y = jnp.dot(h, w2_ref[...], preferred_element_type=jnp.float32) + b2_ref[...]
    bb, l, v = out_ref.shape
    out_ref[...] = y.reshape(bb, LP, v)[:, :l, :]


def kernel(inputs, emb, W1, b1, W2, b2):
    B, L = inputs.shape

    ipad, emb_p, w1_p = _prep(inputs.astype(jnp.int32), emb, W1)

    x = _make_sc_gather(B * LP)(ipad.reshape(B * LP), emb_p)

    out = pl.pallas_call(
        _mlp_kernel,
        grid=(B // _BB,),
        in_specs=[
            pl.BlockSpec((_BB * LP, DP), lambda g: (g, 0)),
            pl.BlockSpec((DP, H), lambda g: (0, 0)),
            pl.BlockSpec((1, H), lambda g: (0, 0)),
            pl.BlockSpec((H, V), lambda g: (0, 0)),
            pl.BlockSpec((1, V), lambda g: (0, 0)),
        ],
        out_specs=pl.BlockSpec((_BB, L, V), lambda g: (g, 0, 0)),
        out_shape=jax.ShapeDtypeStruct((B, L, V), jnp.float32),
    )(x, w1_p, b1.reshape(1, H), W2, b2.reshape(1, V))
    return out
